# Initial kernel scaffold; baseline (speedup 1.0000x reference)
import jax
import jax.numpy as jnp
from jax.experimental import pallas as pl


def kernel(user_id, item_id, language, is_ebook, format, publisher, pub_decade, avg_rating, num_pages, user_table, item_table, language_table, is_ebook_table, format_table, publisher_table, pub_decade_table, Wu1, bu1, Wu2, bu2, Wi1, bi1, Wi2, bi2):
    raise NotImplementedError("write your pallas kernel here")


# speedup vs baseline: 1.3439x; 1.3439x over previous
import jax
import jax.numpy as jnp
from jax import lax
from jax.experimental import pallas as pl
from jax.experimental.pallas import tpu as pltpu
from jax.experimental.pallas import tpu_sc as plsc

B = 16384
D = 32
NC = 2
NS = 16
NW = NC * NS
BPW = B // NW
CH = 128
NCH = BPW // CH

V_BIG = 1000000
UNITS = V_BIG // 128
TAIL0 = UNITS * 128
SLAB_U = 4
SLAB_C = SLAB_U * 128
CAP = 1024
NSEG = B // CAP
NCHUNK = B // 16
HROWS = (B + CAP + 256) // 128
LINR = NW * NSEG * (CAP // 4)
POSR = NW * NSEG * (CAP // 128)
DUMP = B


def _flat16(ref, off):
    return ref[off >> 7, pl.ds(lax.rem(off, 128), 16)]


def _stream_table(t, tt, idx_hbm, lin, pos, cnts, idx_v, hits_i, hits_p,
                  slab_v, stage, pos_st, one_st, wid, lo, hi):
    lo_r = lo * 128
    hi_r = hi * 128
    iota = lax.iota(jnp.int32, 16)
    nslab = (hi - lo + SLAB_U - 1) // SLAB_U

    pltpu.sync_copy(idx_hbm, idx_v)

    def comp_body(si, cnt):
        v = idx_v[si >> 3, pl.ds(lax.rem(si, 8) * 16, 16)]
        m = (v >= lo_r) & (v < hi_r)
        p = cnt + plsc.cumsum(jnp.ones((16,), jnp.int32), mask=m) - 1
        plsc.store_scatter(hits_i, [p >> 7, p & 127], v, mask=m)
        plsc.store_scatter(hits_p, [p >> 7, p & 127], si * 16 + iota, mask=m)
        return cnt + plsc.all_reduce_population_count(m)[0]

    cnt = lax.fori_loop(0, NCHUNK, comp_body, 0)

    for l in range(8):
        one_st[0, pl.ds(l * 16, 16)] = jnp.full((16,), cnt, jnp.int32)
    pltpu.sync_copy(one_st, cnts.at[t, pl.ds(wid, 1)])

    def pad_body(k, _):
        o = cnt + k * 16
        hits_i[o >> 7, pl.ds(lax.rem(o, 128), 16)] = jnp.full(
            (16,), lo_r, jnp.int32)
        hits_p[o >> 7, pl.ds(lax.rem(o, 128), 16)] = jnp.full(
            (16,), DUMP, jnp.int32)
        return ()

    lax.fori_loop(0, 8, pad_body, ())

    def seg_body(b, _):
        base_h = b * CAP
        rem = jnp.minimum(cnt - base_h, CAP)
        nh16 = (rem + 15) // 16

        def slab_body(ts, _):
            s = lo + ts * SLAB_U
            send = jnp.minimum(s + SLAB_U, hi)
            w0 = jnp.minimum(s, hi - SLAB_U)
            pltpu.sync_copy(tt.at[:, pl.ds(w0 * 128, SLAB_C)], slab_v)

            def chunk_body(h, _):
                v = _flat16(hits_i, base_h + h * 16)
                m = (v >= s * 128) & (v < send * 128)
                nhit = plsc.all_reduce_population_count(m)[0]

                @pl.when(nhit > 0)
                def _():
                    local = v - w0 * 128
                    q = (h * 16 + iota) * D
                    for c in range(D):
                        cc = jnp.full((16,), c, jnp.int32)
                        vals = plsc.load_gather(slab_v, [cc, local], mask=m)
                        plsc.store_scatter(stage,
                                           [(q + c) >> 7, (q + c) & 127],
                                           vals, mask=m)
                return ()

            lax.fori_loop(0, nh16, chunk_body, ())
            return ()

        lax.fori_loop(0, nslab, slab_body, ())

        for j in range(CAP // 128):
            for l in range(8):
                pos_st[j, pl.ds(l * 16, 16)] = _flat16(
                    hits_p, base_h + j * 128 + l * 16)
        so = (wid * NSEG + b) * (CAP // 4)
        pltpu.sync_copy(stage, lin.at[t, pl.ds(so, CAP // 4)])
        po = (wid * NSEG + b) * (CAP // 128)
        pltpu.sync_copy(pos_st, pos.at[t, pl.ds(po, CAP // 128)])
        return ()

    lax.fori_loop(0, (cnt + CAP - 1) // CAP, seg_body, ())


def _sc_b1_body(ut, it, uid, iid, lin, pos, cnts,
                idx_v, hits_i, hits_p, slab_v, stage, pos_st, one_st):
    wid = lax.axis_index("s") * NC + lax.axis_index("c")
    lo = (wid * UNITS) // NW
    hi = ((wid + 1) * UNITS) // NW
    _stream_table(0, ut, uid, lin, pos, cnts, idx_v, hits_i, hits_p,
                  slab_v, stage, pos_st, one_st, wid, lo, hi)
    _stream_table(1, it, iid, lin, pos, cnts, idx_v, hits_i, hits_p,
                  slab_v, stage, pos_st, one_st, wid, lo, hi)


def _sc_b1(user_t, item_t, uid2d, iid2d):
    mesh = plsc.VectorSubcoreMesh(core_axis_name="c", subcore_axis_name="s")
    fn = pl.kernel(
        _sc_b1_body, mesh=mesh,
        out_type=[jax.ShapeDtypeStruct((2, LINR, 128), jnp.float32),
                  jax.ShapeDtypeStruct((2, POSR, 128), jnp.int32),
                  jax.ShapeDtypeStruct((2, NW, 128), jnp.int32)],
        scratch_types=[pltpu.VMEM((B // 128, 128), jnp.int32),
                       pltpu.VMEM((HROWS, 128), jnp.int32),
                       pltpu.VMEM((HROWS, 128), jnp.int32),
                       pltpu.VMEM((D, SLAB_C), jnp.float32),
                       pltpu.VMEM((CAP // 4, 128), jnp.float32),
                       pltpu.VMEM((CAP // 128, 128), jnp.int32),
                       pltpu.VMEM((1, 128), jnp.int32)],
        compiler_params=pltpu.CompilerParams(needs_layout_passes=False))
    return fn(user_t.T, item_t.T, uid2d, iid2d)


def _sc_b2_body(lin, pos, cnts, ptab, pidx, uout, iout, pout,
                rows_v, pos_v, cnt_v, idx_v, prow_v, sem):
    wid = lax.axis_index("s") * NC + lax.axis_index("c")

    pltpu.sync_copy(pidx.at[pl.ds(wid * NCH, NCH)], idx_v)
    cps = []
    for j in range(NCH):
        cps.append(pltpu.async_copy(
            ptab.at[idx_v.at[j]], prow_v.at[pl.ds(j * CH, CH)], sem))
    for cp in cps:
        cp.wait()
    pltpu.sync_copy(prow_v, pout.at[pl.ds(wid * BPW, BPW)])

    for t, out in ((0, uout), (1, iout)):
        pltpu.sync_copy(cnts.at[t, pl.ds(wid, 1)], cnt_v)
        cnt = lax.reduce_max(cnt_v[0, pl.ds(0, 16)], axes=(0,))
        nch = (jnp.minimum(cnt, B) + 127) // 128

        def ch_body(j, _):
            b = j >> 3
            r = j & 7
            so = (wid * NSEG + b) * CAP + r * 128
            pltpu.sync_copy(lin.at[t, pl.ds(so, 128)], rows_v)
            po = (wid * NSEG + b) * (CAP // 128) + r
            pltpu.sync_copy(pos.at[t, pl.ds(po, 1)], pos_v)
            pltpu.async_copy(rows_v, out.at[pos_v.at[0]], sem).wait()
            return ()

        lax.fori_loop(0, nch, ch_body, ())


def _sc_b2(lin, pos, cnts, ptab, pidx2d):
    mesh = plsc.VectorSubcoreMesh(core_axis_name="c", subcore_axis_name="s")
    fn = pl.kernel(
        _sc_b2_body, mesh=mesh,
        out_type=[jax.ShapeDtypeStruct((B + 128, D), jnp.float32),
                  jax.ShapeDtypeStruct((B + 128, D), jnp.float32),
                  jax.ShapeDtypeStruct((B, D), jnp.float32)],
        scratch_types=[pltpu.VMEM((128, D), jnp.float32),
                       pltpu.VMEM((1, 128), jnp.int32),
                       pltpu.VMEM((1, 128), jnp.int32),
                       pltpu.VMEM((NCH, CH), jnp.int32),
                       pltpu.VMEM((BPW, D), jnp.float32),
                       pltpu.SemaphoreType.DMA],
        compiler_params=pltpu.CompilerParams(use_tc_tiling_on_sc=False,
                                             needs_layout_passes=False))
    return fn(lin.reshape(2, LINR * 4, D), pos, cnts, ptab, pidx2d)


def _tc_body(ue, ie, pe, uid, iid, lang, ebook, fmt, dec, ex,
             ut_tail, it_tail, lang_t, ebook_t, fmt_t, dec_t,
             wu1, bu1, wu2, bu2, wi1, wex, bi1, wi2, bi2, out):
    hp = jax.lax.Precision.HIGHEST
    f32 = jnp.float32

    def onehot(idx2d, k, base=0):
        io = lax.broadcasted_iota(jnp.int32, (1, k), 1)
        return (idx2d[...] == io + base).astype(f32)

    uef = jnp.where(uid[...] >= TAIL0,
                    jax.lax.dot(onehot(uid, 64, TAIL0), ut_tail[...],
                                precision=hp),
                    ue[...])
    ief = jnp.where(iid[...] >= TAIL0,
                    jax.lax.dot(onehot(iid, 64, TAIL0), it_tail[...],
                                precision=hp),
                    ie[...])
    hu = jax.lax.dot(uef, wu1[...], precision=hp) + bu1[...]
    hu = hu * jax.nn.sigmoid(hu)
    u = jax.lax.dot(hu, wu2[...], precision=hp) + bu2[...]
    w = wi1[...]
    dot = lambda a, b: jax.lax.dot(a, b, precision=hp)
    hi = dot(ief, w[0:D])
    hi = hi + dot(dot(onehot(lang, 128), lang_t[...]), w[D:2 * D])
    hi = hi + dot(dot(onehot(ebook, 8), ebook_t[...]), w[2 * D:3 * D])
    hi = hi + dot(dot(onehot(fmt, 32), fmt_t[...]), w[3 * D:4 * D])
    hi = hi + dot(pe[...], w[4 * D:5 * D])
    hi = hi + dot(dot(onehot(dec, 32), dec_t[...]), w[5 * D:6 * D])
    hi = hi + dot(ex[...], wex[...]) + bi1[...]
    hi = hi * jax.nn.sigmoid(hi)
    iv = jax.lax.dot(hi, wi2[...], precision=hp) + bi2[...]
    out[...] = jnp.sum(u * iv, axis=1, keepdims=True)


def _tc_dense(ue, ie, pe, uid, iid, lang, ebook, fmt, dec, ex,
              ut_tail, it_tail, lang_t, ebook_t, fmt_t, dec_t,
              Wu1, bu1, Wu2, bu2, Wi1, Wex, bi1, Wi2, bi2):
    bs = 2048
    grid = (B // bs,)
    row = lambda i: (i, 0)
    fix = lambda i: (0, 0)
    emb = pl.BlockSpec((bs, D), row)
    idxs = pl.BlockSpec((bs, 1), row)
    in_specs = [
        emb, emb, emb,
        idxs, idxs, idxs, idxs, idxs, idxs,
        pl.BlockSpec((bs, 2), row),
        pl.BlockSpec((64, D), fix), pl.BlockSpec((64, D), fix),
        pl.BlockSpec((128, D), fix), pl.BlockSpec((8, D), fix),
        pl.BlockSpec((32, D), fix), pl.BlockSpec((32, D), fix),
        pl.BlockSpec((D, D), fix), pl.BlockSpec((1, D), fix),
        pl.BlockSpec((D, D), fix), pl.BlockSpec((1, D), fix),
        pl.BlockSpec((6 * D, D), fix), pl.BlockSpec((2, D), fix),
        pl.BlockSpec((1, D), fix),
        pl.BlockSpec((D, D), fix), pl.BlockSpec((1, D), fix),
    ]
    out = pl.pallas_call(
        _tc_body,
        grid=grid,
        in_specs=in_specs,
        out_specs=pl.BlockSpec((bs, 1), row),
        out_shape=jax.ShapeDtypeStruct((B, 1), jnp.float32),
    )(ue, ie, pe, uid, iid, lang, ebook, fmt, dec, ex,
      ut_tail, it_tail, lang_t, ebook_t, fmt_t, dec_t,
      Wu1, bu1, Wu2, bu2, Wi1, Wex, bi1, Wi2, bi2)
    return out.reshape(B)


def kernel(user_id, item_id, language, is_ebook, format, publisher, pub_decade,
           avg_rating, num_pages,
           user_table, item_table, language_table, is_ebook_table, format_table,
           publisher_table, pub_decade_table,
           Wu1, bu1, Wu2, bu2, Wi1, bi1, Wi2, bi2):
    i32 = jnp.int32
    uid = user_id.astype(i32)
    iid = item_id.astype(i32)
    lin, pos, cnts = _sc_b1(user_table, item_table,
                            uid.reshape(B // 128, 128),
                            iid.reshape(B // 128, 128))
    ue, ie, pe = _sc_b2(lin, pos, cnts, publisher_table,
                        publisher.astype(i32).reshape(B // CH, CH))
    ex = jnp.stack([avg_rating, num_pages], axis=1)
    pad = lambda t, k: jnp.pad(t, ((0, k - t.shape[0]), (0, 0)))
    return _tc_dense(
        ue[:B], ie[:B], pe,
        uid.reshape(B, 1), iid.reshape(B, 1),
        language.astype(i32).reshape(B, 1), is_ebook.astype(i32).reshape(B, 1),
        format.astype(i32).reshape(B, 1), pub_decade.astype(i32).reshape(B, 1),
        ex,
        user_table[TAIL0:], item_table[TAIL0:],
        pad(language_table, 128), pad(is_ebook_table, 8),
        pad(format_table, 32), pad(pub_decade_table, 32),
        Wu1, bu1.reshape(1, D), Wu2, bu2.reshape(1, D),
        Wi1[:6 * D], Wi1[6 * D:], bi1.reshape(1, D),
        Wi2, bi2.reshape(1, D))


# speedup vs baseline: 1.6911x; 1.2584x over previous
import jax
import jax.numpy as jnp
from jax import lax
from jax.experimental import pallas as pl
from jax.experimental.pallas import tpu as pltpu
from jax.experimental.pallas import tpu_sc as plsc

B = 16384
D = 32
NC = 2
NS = 16
NW = NC * NS
BPW = B // NW
CH = 128
NCH = BPW // CH

V_BIG = 1000000
UNITS = V_BIG // 128
TAIL0 = UNITS * 128
SLAB_U = 4
SLAB_C = SLAB_U * 128
CAP = 1024
NSEG = B // CAP
NCHUNK = B // 16
HROWS = (B + CAP + 256) // 128
LINR = NW * NSEG * (CAP // 4)
POSR = NW * NSEG * (CAP // 128)
DUMP = B


def _flat16(ref, off):
    return ref[off >> 7, pl.ds(lax.rem(off, 128), 16)]


def _stream_table(t, tt, idx_hbm, lin, pos, cnts, idx_v, hits_i, hits_p,
                  slab_v, slab_w, stage, pos_st, one_st, sem_a, sem_b,
                  wid, lo, hi):
    lo_r = lo * 128
    hi_r = hi * 128
    iota = lax.iota(jnp.int32, 16)
    nslab = (hi - lo + SLAB_U - 1) // SLAB_U

    pltpu.sync_copy(idx_hbm, idx_v)

    def comp_body(si, cnt):
        v = idx_v[si >> 3, pl.ds(lax.rem(si, 8) * 16, 16)]
        m = (v >= lo_r) & (v < hi_r)
        p = cnt + plsc.cumsum(jnp.ones((16,), jnp.int32), mask=m) - 1
        plsc.store_scatter(hits_i, [p >> 7, p & 127], v, mask=m)
        plsc.store_scatter(hits_p, [p >> 7, p & 127], si * 16 + iota, mask=m)
        return cnt + plsc.all_reduce_population_count(m)[0]

    cnt = lax.fori_loop(0, NCHUNK, comp_body, 0)

    for l in range(8):
        one_st[0, pl.ds(l * 16, 16)] = jnp.full((16,), cnt, jnp.int32)
    pltpu.sync_copy(one_st, cnts.at[t, pl.ds(wid, 1)])

    def pad_body(k, _):
        o = cnt + k * 16
        hits_i[o >> 7, pl.ds(lax.rem(o, 128), 16)] = jnp.full(
            (16,), lo_r, jnp.int32)
        hits_p[o >> 7, pl.ds(lax.rem(o, 128), 16)] = jnp.full(
            (16,), DUMP, jnp.int32)
        return ()

    lax.fori_loop(0, 8, pad_body, ())

    def seg_body(b, _):
        base_h = b * CAP
        rem = jnp.minimum(cnt - base_h, CAP)
        nh16 = (rem + 15) // 16

        def w0_of(ts):
            s = jnp.minimum(lo + ts * SLAB_U, hi - SLAB_U)
            return jnp.maximum(s, lo)

        def issue(ts, buf, sem):
            pltpu.async_copy(tt.at[:, pl.ds(w0_of(ts) * 128, SLAB_C)],
                             buf, sem)

        def drain(buf, sem):
            pltpu.make_async_copy(tt.at[:, pl.ds(0, SLAB_C)], buf, sem).wait()

        def scan(ts, buf):
            s = lo + ts * SLAB_U
            send = jnp.minimum(s + SLAB_U, hi)
            w0 = w0_of(ts)

            def chunk_body(h, _):
                v = _flat16(hits_i, base_h + h * 16)
                m = (v >= s * 128) & (v < send * 128)
                nhit = plsc.all_reduce_population_count(m)[0]

                @pl.when(nhit > 0)
                def _():
                    local = v - w0 * 128
                    q = (h * 16 + iota) * D
                    for c in range(D):
                        cc = jnp.full((16,), c, jnp.int32)
                        vals = plsc.load_gather(buf, [cc, local], mask=m)
                        plsc.store_scatter(stage,
                                           [(q + c) >> 7, (q + c) & 127],
                                           vals, mask=m)
                return ()

            lax.fori_loop(0, nh16, chunk_body, ())

        issue(0, slab_v, sem_a)

        def pair_body(g, _):
            ts0 = 2 * g
            issue(ts0 + 1, slab_w, sem_b)
            drain(slab_v, sem_a)
            scan(ts0, slab_v)
            issue(ts0 + 2, slab_v, sem_a)
            drain(slab_w, sem_b)
            scan(ts0 + 1, slab_w)
            return ()

        lax.fori_loop(0, (nslab + 1) // 2, pair_body, ())
        drain(slab_v, sem_a)

        for j in range(CAP // 128):
            for l in range(8):
                pos_st[j, pl.ds(l * 16, 16)] = _flat16(
                    hits_p, base_h + j * 128 + l * 16)
        so = (wid * NSEG + b) * (CAP // 4)
        pltpu.sync_copy(stage, lin.at[t, pl.ds(so, CAP // 4)])
        po = (wid * NSEG + b) * (CAP // 128)
        pltpu.sync_copy(pos_st, pos.at[t, pl.ds(po, CAP // 128)])
        return ()

    lax.fori_loop(0, (cnt + CAP - 1) // CAP, seg_body, ())


def _sc_b1_body(ut, it, uid, iid, lin, pos, cnts,
                idx_v, hits_i, hits_p, slab_v, slab_w, stage, pos_st, one_st,
                sem_a, sem_b):
    wid = lax.axis_index("s") * NC + lax.axis_index("c")
    lo = (wid * UNITS) // NW
    hi = ((wid + 1) * UNITS) // NW
    _stream_table(0, ut, uid, lin, pos, cnts, idx_v, hits_i, hits_p,
                  slab_v, slab_w, stage, pos_st, one_st, sem_a, sem_b,
                  wid, lo, hi)
    _stream_table(1, it, iid, lin, pos, cnts, idx_v, hits_i, hits_p,
                  slab_v, slab_w, stage, pos_st, one_st, sem_a, sem_b,
                  wid, lo, hi)


def _sc_b1(user_t, item_t, uid2d, iid2d):
    mesh = plsc.VectorSubcoreMesh(core_axis_name="c", subcore_axis_name="s")
    fn = pl.kernel(
        _sc_b1_body, mesh=mesh,
        out_type=[jax.ShapeDtypeStruct((2, LINR, 128), jnp.float32),
                  jax.ShapeDtypeStruct((2, POSR, 128), jnp.int32),
                  jax.ShapeDtypeStruct((2, NW, 128), jnp.int32)],
        scratch_types=[pltpu.VMEM((B // 128, 128), jnp.int32),
                       pltpu.VMEM((HROWS, 128), jnp.int32),
                       pltpu.VMEM((HROWS, 128), jnp.int32),
                       pltpu.VMEM((D, SLAB_C), jnp.float32),
                       pltpu.VMEM((D, SLAB_C), jnp.float32),
                       pltpu.VMEM((CAP // 4, 128), jnp.float32),
                       pltpu.VMEM((CAP // 128, 128), jnp.int32),
                       pltpu.VMEM((1, 128), jnp.int32),
                       pltpu.SemaphoreType.DMA,
                       pltpu.SemaphoreType.DMA],
        compiler_params=pltpu.CompilerParams(needs_layout_passes=False))
    return fn(user_t.T, item_t.T, uid2d, iid2d)


def _sc_b2_body(lin, pos, cnts, ptab, pidx, uout, iout, pout,
                rows_v, pos_v, cnt_v, idx_v, prow_v, sem):
    wid = lax.axis_index("s") * NC + lax.axis_index("c")

    pltpu.sync_copy(pidx.at[pl.ds(wid * NCH, NCH)], idx_v)
    cps = []
    for j in range(NCH):
        cps.append(pltpu.async_copy(
            ptab.at[idx_v.at[j]], prow_v.at[pl.ds(j * CH, CH)], sem))
    for cp in cps:
        cp.wait()
    pltpu.sync_copy(prow_v, pout.at[pl.ds(wid * BPW, BPW)])

    for t, out in ((0, uout), (1, iout)):
        pltpu.sync_copy(cnts.at[t, pl.ds(wid, 1)], cnt_v)
        cnt = lax.reduce_max(cnt_v[0, pl.ds(0, 16)], axes=(0,))
        nch = (jnp.minimum(cnt, B) + 127) // 128

        def ch_body(j, _):
            b = j >> 3
            r = j & 7
            so = (wid * NSEG + b) * CAP + r * 128
            pltpu.sync_copy(lin.at[t, pl.ds(so, 128)], rows_v)
            po = (wid * NSEG + b) * (CAP // 128) + r
            pltpu.sync_copy(pos.at[t, pl.ds(po, 1)], pos_v)
            pltpu.async_copy(rows_v, out.at[pos_v.at[0]], sem).wait()
            return ()

        lax.fori_loop(0, nch, ch_body, ())


def _sc_b2(lin, pos, cnts, ptab, pidx2d):
    mesh = plsc.VectorSubcoreMesh(core_axis_name="c", subcore_axis_name="s")
    fn = pl.kernel(
        _sc_b2_body, mesh=mesh,
        out_type=[jax.ShapeDtypeStruct((B + 128, D), jnp.float32),
                  jax.ShapeDtypeStruct((B + 128, D), jnp.float32),
                  jax.ShapeDtypeStruct((B, D), jnp.float32)],
        scratch_types=[pltpu.VMEM((128, D), jnp.float32),
                       pltpu.VMEM((1, 128), jnp.int32),
                       pltpu.VMEM((1, 128), jnp.int32),
                       pltpu.VMEM((NCH, CH), jnp.int32),
                       pltpu.VMEM((BPW, D), jnp.float32),
                       pltpu.SemaphoreType.DMA],
        compiler_params=pltpu.CompilerParams(use_tc_tiling_on_sc=False,
                                             needs_layout_passes=False))
    return fn(lin.reshape(2, LINR * 4, D), pos, cnts, ptab, pidx2d)


def _tc_body(ue, ie, pe, uid, iid, lang, ebook, fmt, dec, ex,
             ut_tail, it_tail, lang_t, ebook_t, fmt_t, dec_t,
             wu1, bu1, wu2, bu2, wi1, wex, bi1, wi2, bi2, out):
    hp = jax.lax.Precision.HIGHEST
    f32 = jnp.float32

    def onehot(idx2d, k, base=0):
        io = lax.broadcasted_iota(jnp.int32, (1, k), 1)
        return (idx2d[...] == io + base).astype(f32)

    uef = jnp.where(uid[...] >= TAIL0,
                    jax.lax.dot(onehot(uid, 64, TAIL0), ut_tail[...],
                                precision=hp),
                    ue[...])
    ief = jnp.where(iid[...] >= TAIL0,
                    jax.lax.dot(onehot(iid, 64, TAIL0), it_tail[...],
                                precision=hp),
                    ie[...])
    hu = jax.lax.dot(uef, wu1[...], precision=hp) + bu1[...]
    hu = hu * jax.nn.sigmoid(hu)
    u = jax.lax.dot(hu, wu2[...], precision=hp) + bu2[...]
    w = wi1[...]
    dot = lambda a, b: jax.lax.dot(a, b, precision=hp)
    hi = dot(ief, w[0:D])
    hi = hi + dot(dot(onehot(lang, 128), lang_t[...]), w[D:2 * D])
    hi = hi + dot(dot(onehot(ebook, 8), ebook_t[...]), w[2 * D:3 * D])
    hi = hi + dot(dot(onehot(fmt, 32), fmt_t[...]), w[3 * D:4 * D])
    hi = hi + dot(pe[...], w[4 * D:5 * D])
    hi = hi + dot(dot(onehot(dec, 32), dec_t[...]), w[5 * D:6 * D])
    hi = hi + dot(ex[...], wex[...]) + bi1[...]
    hi = hi * jax.nn.sigmoid(hi)
    iv = jax.lax.dot(hi, wi2[...], precision=hp) + bi2[...]
    out[...] = jnp.sum(u * iv, axis=1, keepdims=True)


def _tc_dense(ue, ie, pe, uid, iid, lang, ebook, fmt, dec, ex,
              ut_tail, it_tail, lang_t, ebook_t, fmt_t, dec_t,
              Wu1, bu1, Wu2, bu2, Wi1, Wex, bi1, Wi2, bi2):
    bs = 2048
    grid = (B // bs,)
    row = lambda i: (i, 0)
    fix = lambda i: (0, 0)
    emb = pl.BlockSpec((bs, D), row)
    idxs = pl.BlockSpec((bs, 1), row)
    in_specs = [
        emb, emb, emb,
        idxs, idxs, idxs, idxs, idxs, idxs,
        pl.BlockSpec((bs, 2), row),
        pl.BlockSpec((64, D), fix), pl.BlockSpec((64, D), fix),
        pl.BlockSpec((128, D), fix), pl.BlockSpec((8, D), fix),
        pl.BlockSpec((32, D), fix), pl.BlockSpec((32, D), fix),
        pl.BlockSpec((D, D), fix), pl.BlockSpec((1, D), fix),
        pl.BlockSpec((D, D), fix), pl.BlockSpec((1, D), fix),
        pl.BlockSpec((6 * D, D), fix), pl.BlockSpec((2, D), fix),
        pl.BlockSpec((1, D), fix),
        pl.BlockSpec((D, D), fix), pl.BlockSpec((1, D), fix),
    ]
    out = pl.pallas_call(
        _tc_body,
        grid=grid,
        in_specs=in_specs,
        out_specs=pl.BlockSpec((bs, 1), row),
        out_shape=jax.ShapeDtypeStruct((B, 1), jnp.float32),
    )(ue, ie, pe, uid, iid, lang, ebook, fmt, dec, ex,
      ut_tail, it_tail, lang_t, ebook_t, fmt_t, dec_t,
      Wu1, bu1, Wu2, bu2, Wi1, Wex, bi1, Wi2, bi2)
    return out.reshape(B)


def kernel(user_id, item_id, language, is_ebook, format, publisher, pub_decade,
           avg_rating, num_pages,
           user_table, item_table, language_table, is_ebook_table, format_table,
           publisher_table, pub_decade_table,
           Wu1, bu1, Wu2, bu2, Wi1, bi1, Wi2, bi2):
    i32 = jnp.int32
    uid = user_id.astype(i32)
    iid = item_id.astype(i32)
    lin, pos, cnts = _sc_b1(user_table, item_table,
                            uid.reshape(B // 128, 128),
                            iid.reshape(B // 128, 128))
    ue, ie, pe = _sc_b2(lin, pos, cnts, publisher_table,
                        publisher.astype(i32).reshape(B // CH, CH))
    ex = jnp.stack([avg_rating, num_pages], axis=1)
    pad = lambda t, k: jnp.pad(t, ((0, k - t.shape[0]), (0, 0)))
    return _tc_dense(
        ue[:B], ie[:B], pe,
        uid.reshape(B, 1), iid.reshape(B, 1),
        language.astype(i32).reshape(B, 1), is_ebook.astype(i32).reshape(B, 1),
        format.astype(i32).reshape(B, 1), pub_decade.astype(i32).reshape(B, 1),
        ex,
        user_table[TAIL0:], item_table[TAIL0:],
        pad(language_table, 128), pad(is_ebook_table, 8),
        pad(format_table, 32), pad(pub_decade_table, 32),
        Wu1, bu1.reshape(1, D), Wu2, bu2.reshape(1, D),
        Wi1[:6 * D], Wi1[6 * D:], bi1.reshape(1, D),
        Wi2, bi2.reshape(1, D))


# speedup vs baseline: 1.8908x; 1.1181x over previous
import jax
import jax.numpy as jnp
from jax import lax
from jax.experimental import pallas as pl
from jax.experimental.pallas import tpu as pltpu
from jax.experimental.pallas import tpu_sc as plsc

B = 16384
D = 32
NC = 2
NS = 16
NW = NC * NS
BPW = B // NW
CH = 128
NCH = BPW // CH

V_BIG = 1000000
UNITS = V_BIG // 128
TAIL0 = UNITS * 128
SLAB_U = 8
SLAB_C = SLAB_U * 128
CAP = 768
NSEG = -(B // -CAP)
NCHUNK = B // 16
HROWS = (B + CAP + 256) // 128
LINR = NW * NSEG * (CAP // 4)
POSR = NW * NSEG * 8
DUMP = B


def _flat16(ref, off):
    return ref[off >> 7, pl.ds(lax.rem(off, 128), 16)]


def _stream_table(t, tt, idx_hbm, lin, pos, cnts, idx_v, hits_i, hits_p,
                  slab_v, slab_w, stage, pos_st, one_st, sem_a, sem_b,
                  wid, lo, hi):
    lo_r = lo * 128
    hi_r = hi * 128
    iota = lax.iota(jnp.int32, 16)
    nslab = (hi - lo + SLAB_U - 1) // SLAB_U

    def comp_outer(ii, cnt):
        pltpu.sync_copy(idx_hbm.at[pl.ds(ii * 16, 16)], idx_v)

        def comp_body(si, cnt):
            v = idx_v[si >> 3, pl.ds(lax.rem(si, 8) * 16, 16)]
            m = (v >= lo_r) & (v < hi_r)
            p = cnt + plsc.cumsum(jnp.ones((16,), jnp.int32), mask=m) - 1
            plsc.store_scatter(hits_i, [p >> 7, p & 127], v, mask=m)
            plsc.store_scatter(hits_p, [p >> 7, p & 127],
                               (ii * 128 + si) * 16 + iota, mask=m)
            return cnt + plsc.all_reduce_population_count(m)[0]

        return lax.fori_loop(0, 128, comp_body, cnt)

    cnt = lax.fori_loop(0, B // 128 // 16, comp_outer, 0)

    for l in range(8):
        one_st[0, pl.ds(l * 16, 16)] = jnp.full((16,), cnt, jnp.int32)
    pltpu.sync_copy(one_st, cnts.at[t, pl.ds(wid, 1)])

    def pad_body(k, _):
        o = cnt + k * 16
        hits_i[o >> 7, pl.ds(lax.rem(o, 128), 16)] = jnp.full(
            (16,), lo_r, jnp.int32)
        hits_p[o >> 7, pl.ds(lax.rem(o, 128), 16)] = jnp.full(
            (16,), DUMP, jnp.int32)
        return ()

    lax.fori_loop(0, 8, pad_body, ())

    def seg_body(b, _):
        base_h = b * CAP
        rem = jnp.minimum(cnt - base_h, CAP)
        nh16 = (rem + 15) // 16

        def w0_of(ts):
            s = jnp.minimum(lo + ts * SLAB_U, hi - SLAB_U)
            return jnp.maximum(s, lo)

        def issue(ts, buf, sem):
            pltpu.async_copy(tt.at[:, pl.ds(w0_of(ts) * 128, SLAB_C)],
                             buf, sem)

        def drain(buf, sem):
            pltpu.make_async_copy(tt.at[:, pl.ds(0, SLAB_C)], buf, sem).wait()

        def scan(ts, buf):
            s = lo + ts * SLAB_U
            send = jnp.minimum(s + SLAB_U, hi)
            w0 = w0_of(ts)

            def chunk_body(h, _):
                v = _flat16(hits_i, base_h + h * 16)
                m = (v >= s * 128) & (v < send * 128)
                nhit = plsc.all_reduce_population_count(m)[0]

                @pl.when(nhit > 0)
                def _():
                    local = v - w0 * 128
                    q = (h * 16 + iota) * D
                    for c in range(D):
                        cc = jnp.full((16,), c, jnp.int32)
                        vals = plsc.load_gather(buf, [cc, local], mask=m)
                        plsc.store_scatter(stage,
                                           [(q + c) >> 7, (q + c) & 127],
                                           vals, mask=m)
                return ()

            lax.fori_loop(0, nh16, chunk_body, ())

        issue(0, slab_v, sem_a)

        def pair_body(g, _):
            ts0 = 2 * g
            issue(ts0 + 1, slab_w, sem_b)
            drain(slab_v, sem_a)
            scan(ts0, slab_v)
            issue(ts0 + 2, slab_v, sem_a)
            drain(slab_w, sem_b)
            scan(ts0 + 1, slab_w)
            return ()

        lax.fori_loop(0, (nslab + 1) // 2, pair_body, ())
        drain(slab_v, sem_a)

        for j in range(CAP // 128):
            for l in range(8):
                pos_st[j, pl.ds(l * 16, 16)] = _flat16(
                    hits_p, base_h + j * 128 + l * 16)
        so = (wid * NSEG + b) * (CAP // 4)
        pltpu.sync_copy(stage, lin.at[t, pl.ds(so, CAP // 4)])
        po = (wid * NSEG + b) * 8
        pltpu.sync_copy(pos_st, pos.at[t, pl.ds(po, 8)])
        return ()

    lax.fori_loop(0, (cnt + CAP - 1) // CAP, seg_body, ())


def _sc_b1_body(ut, it, uid, iid, lin, pos, cnts,
                idx_v, hits_i, hits_p, slab_v, slab_w, stage, pos_st, one_st,
                sem_a, sem_b):
    wid = lax.axis_index("s") * NC + lax.axis_index("c")
    lo = (wid * UNITS) // NW
    hi = ((wid + 1) * UNITS) // NW
    _stream_table(0, ut, uid, lin, pos, cnts, idx_v, hits_i, hits_p,
                  slab_v, slab_w, stage, pos_st, one_st, sem_a, sem_b,
                  wid, lo, hi)
    _stream_table(1, it, iid, lin, pos, cnts, idx_v, hits_i, hits_p,
                  slab_v, slab_w, stage, pos_st, one_st, sem_a, sem_b,
                  wid, lo, hi)


def _sc_b1(user_t, item_t, uid2d, iid2d):
    mesh = plsc.VectorSubcoreMesh(core_axis_name="c", subcore_axis_name="s")
    fn = pl.kernel(
        _sc_b1_body, mesh=mesh,
        out_type=[jax.ShapeDtypeStruct((2, LINR, 128), jnp.float32),
                  jax.ShapeDtypeStruct((2, POSR, 128), jnp.int32),
                  jax.ShapeDtypeStruct((2, NW, 128), jnp.int32)],
        scratch_types=[pltpu.VMEM((16, 128), jnp.int32),
                       pltpu.VMEM((HROWS, 128), jnp.int32),
                       pltpu.VMEM((HROWS, 128), jnp.int32),
                       pltpu.VMEM((D, SLAB_C), jnp.float32),
                       pltpu.VMEM((D, SLAB_C), jnp.float32),
                       pltpu.VMEM((CAP // 4, 128), jnp.float32),
                       pltpu.VMEM((8, 128), jnp.int32),
                       pltpu.VMEM((1, 128), jnp.int32),
                       pltpu.SemaphoreType.DMA,
                       pltpu.SemaphoreType.DMA],
        compiler_params=pltpu.CompilerParams(needs_layout_passes=False))
    return fn(user_t.T, item_t.T, uid2d, iid2d)


def _sc_b2_body(lin, pos, cnts, ptab, pidx, uout, iout, pout,
                rows_v, pos_v, cnt_v, idx_v, prow_v, sem):
    wid = lax.axis_index("s") * NC + lax.axis_index("c")

    pltpu.sync_copy(pidx.at[pl.ds(wid * NCH, NCH)], idx_v)
    cps = []
    for j in range(NCH):
        cps.append(pltpu.async_copy(
            ptab.at[idx_v.at[j]], prow_v.at[pl.ds(j * CH, CH)], sem))
    for cp in cps:
        cp.wait()
    pltpu.sync_copy(prow_v, pout.at[pl.ds(wid * BPW, BPW)])

    for t, out in ((0, uout), (1, iout)):
        pltpu.sync_copy(cnts.at[t, pl.ds(wid, 1)], cnt_v)
        cnt = lax.reduce_max(cnt_v[0, pl.ds(0, 16)], axes=(0,))
        nch = (jnp.minimum(cnt, B) + 127) // 128

        def ch_body(j, _):
            so = wid * NSEG * CAP + j * 128
            pltpu.sync_copy(lin.at[t, pl.ds(so, 128)], rows_v)
            nseg_c = CAP // 128
            po = (wid * NSEG + j // nseg_c) * 8 + lax.rem(j, nseg_c)
            pltpu.sync_copy(pos.at[t, pl.ds(po, 1)], pos_v)
            pltpu.async_copy(rows_v, out.at[pos_v.at[0]], sem).wait()
            return ()

        lax.fori_loop(0, nch, ch_body, ())


def _sc_b2(lin, pos, cnts, ptab, pidx2d):
    mesh = plsc.VectorSubcoreMesh(core_axis_name="c", subcore_axis_name="s")
    fn = pl.kernel(
        _sc_b2_body, mesh=mesh,
        out_type=[jax.ShapeDtypeStruct((B + 128, D), jnp.float32),
                  jax.ShapeDtypeStruct((B + 128, D), jnp.float32),
                  jax.ShapeDtypeStruct((B, D), jnp.float32)],
        scratch_types=[pltpu.VMEM((128, D), jnp.float32),
                       pltpu.VMEM((1, 128), jnp.int32),
                       pltpu.VMEM((1, 128), jnp.int32),
                       pltpu.VMEM((NCH, CH), jnp.int32),
                       pltpu.VMEM((BPW, D), jnp.float32),
                       pltpu.SemaphoreType.DMA],
        compiler_params=pltpu.CompilerParams(use_tc_tiling_on_sc=False,
                                             needs_layout_passes=False))
    return fn(lin.reshape(2, LINR * 4, D), pos, cnts, ptab, pidx2d)


def _tc_body(ue, ie, pe, uid, iid, lang, ebook, fmt, dec, ex,
             ut_tail, it_tail, lang_t, ebook_t, fmt_t, dec_t,
             wu1, bu1, wu2, bu2, wi1, wex, bi1, wi2, bi2, out):
    hp = jax.lax.Precision.HIGHEST
    f32 = jnp.float32

    def onehot(idx2d, k, base=0):
        io = lax.broadcasted_iota(jnp.int32, (1, k), 1)
        return (idx2d[...] == io + base).astype(f32)

    uef = jnp.where(uid[...] >= TAIL0,
                    jax.lax.dot(onehot(uid, 64, TAIL0), ut_tail[...],
                                precision=hp),
                    ue[...])
    ief = jnp.where(iid[...] >= TAIL0,
                    jax.lax.dot(onehot(iid, 64, TAIL0), it_tail[...],
                                precision=hp),
                    ie[...])
    hu = jax.lax.dot(uef, wu1[...], precision=hp) + bu1[...]
    hu = hu * jax.nn.sigmoid(hu)
    u = jax.lax.dot(hu, wu2[...], precision=hp) + bu2[...]
    w = wi1[...]
    dot = lambda a, b: jax.lax.dot(a, b, precision=hp)
    hi = dot(ief, w[0:D])
    hi = hi + dot(dot(onehot(lang, 128), lang_t[...]), w[D:2 * D])
    hi = hi + dot(dot(onehot(ebook, 8), ebook_t[...]), w[2 * D:3 * D])
    hi = hi + dot(dot(onehot(fmt, 32), fmt_t[...]), w[3 * D:4 * D])
    hi = hi + dot(pe[...], w[4 * D:5 * D])
    hi = hi + dot(dot(onehot(dec, 32), dec_t[...]), w[5 * D:6 * D])
    hi = hi + dot(ex[...], wex[...]) + bi1[...]
    hi = hi * jax.nn.sigmoid(hi)
    iv = jax.lax.dot(hi, wi2[...], precision=hp) + bi2[...]
    out[...] = jnp.sum(u * iv, axis=1, keepdims=True)


def _tc_dense(ue, ie, pe, uid, iid, lang, ebook, fmt, dec, ex,
              ut_tail, it_tail, lang_t, ebook_t, fmt_t, dec_t,
              Wu1, bu1, Wu2, bu2, Wi1, Wex, bi1, Wi2, bi2):
    bs = 2048
    grid = (B // bs,)
    row = lambda i: (i, 0)
    fix = lambda i: (0, 0)
    emb = pl.BlockSpec((bs, D), row)
    idxs = pl.BlockSpec((bs, 1), row)
    in_specs = [
        emb, emb, emb,
        idxs, idxs, idxs, idxs, idxs, idxs,
        pl.BlockSpec((bs, 2), row),
        pl.BlockSpec((64, D), fix), pl.BlockSpec((64, D), fix),
        pl.BlockSpec((128, D), fix), pl.BlockSpec((8, D), fix),
        pl.BlockSpec((32, D), fix), pl.BlockSpec((32, D), fix),
        pl.BlockSpec((D, D), fix), pl.BlockSpec((1, D), fix),
        pl.BlockSpec((D, D), fix), pl.BlockSpec((1, D), fix),
        pl.BlockSpec((6 * D, D), fix), pl.BlockSpec((2, D), fix),
        pl.BlockSpec((1, D), fix),
        pl.BlockSpec((D, D), fix), pl.BlockSpec((1, D), fix),
    ]
    out = pl.pallas_call(
        _tc_body,
        grid=grid,
        in_specs=in_specs,
        out_specs=pl.BlockSpec((bs, 1), row),
        out_shape=jax.ShapeDtypeStruct((B, 1), jnp.float32),
    )(ue, ie, pe, uid, iid, lang, ebook, fmt, dec, ex,
      ut_tail, it_tail, lang_t, ebook_t, fmt_t, dec_t,
      Wu1, bu1, Wu2, bu2, Wi1, Wex, bi1, Wi2, bi2)
    return out.reshape(B)


def kernel(user_id, item_id, language, is_ebook, format, publisher, pub_decade,
           avg_rating, num_pages,
           user_table, item_table, language_table, is_ebook_table, format_table,
           publisher_table, pub_decade_table,
           Wu1, bu1, Wu2, bu2, Wi1, bi1, Wi2, bi2):
    i32 = jnp.int32
    uid = user_id.astype(i32)
    iid = item_id.astype(i32)
    lin, pos, cnts = _sc_b1(user_table, item_table,
                            uid.reshape(B // 128, 128),
                            iid.reshape(B // 128, 128))
    ue, ie, pe = _sc_b2(lin, pos, cnts, publisher_table,
                        publisher.astype(i32).reshape(B // CH, CH))
    ex = jnp.stack([avg_rating, num_pages], axis=1)
    pad = lambda t, k: jnp.pad(t, ((0, k - t.shape[0]), (0, 0)))
    return _tc_dense(
        ue[:B], ie[:B], pe,
        uid.reshape(B, 1), iid.reshape(B, 1),
        language.astype(i32).reshape(B, 1), is_ebook.astype(i32).reshape(B, 1),
        format.astype(i32).reshape(B, 1), pub_decade.astype(i32).reshape(B, 1),
        ex,
        user_table[TAIL0:], item_table[TAIL0:],
        pad(language_table, 128), pad(is_ebook_table, 8),
        pad(format_table, 32), pad(pub_decade_table, 32),
        Wu1, bu1.reshape(1, D), Wu2, bu2.reshape(1, D),
        Wi1[:6 * D], Wi1[6 * D:], bi1.reshape(1, D),
        Wi2, bi2.reshape(1, D))
